# baseline (speedup 1.0000x reference)
import functools

import jax
import jax.numpy as jnp
from jax import lax
from jax.experimental import pallas as pl
from jax.experimental.pallas import tpu as pltpu
from jax.experimental.pallas import tpu_sc as plsc

N = 10000
E = 320000
D_FEAT = 128
D_HID = 64
D_COM = 8
D_TYPE = 8
H = 32
NCLS = 4

NC = 2
NS = 16
LANES = 16
NW = NC * NS
E_PER_W = E // NW


NB = 2000
G = N // NB


def _tc_body(nf, nh, cv, te, nh_full, We1, be1, We2, be2, Wn1, bn1, Wn2,
             bn2, sd_out, np_out, w_s, bias_s):
    f32 = jnp.float32

    def mm(a, b):
        return jax.lax.dot(a, b, precision=jax.lax.Precision.HIGHEST,
                           preferred_element_type=f32)

    @pl.when(pl.program_id(0) == 0)
    def _prep():
        We2v = We2[:]
        Wn2v = Wn2[:]
        mean = jnp.mean(nh_full[:], axis=0, keepdims=True)

        w_s[0:128, :] = jnp.concatenate(
            [mm(We1[0:128, :], We2v), mm(We1[208:336, :], We2v),
             mm(Wn1[0:128, :], Wn2v)], axis=1)
        w_s[128:192, :] = jnp.concatenate(
            [mm(We1[128:192, :], We2v), mm(We1[336:400, :], We2v),
             mm(Wn1[128:192, :], Wn2v)], axis=1)
        w_s[192:200, :] = jnp.concatenate(
            [mm(We1[192:200, :], We2v), mm(We1[200:208, :], We2v),
             mm(Wn1[264:272, :], Wn2v)], axis=1)
        w_s[200:208, :] = jnp.concatenate(
            [mm(We1[408:416, :], We2v), mm(We1[400:408, :], We2v),
             mm(Wn1[256:264, :], Wn2v)], axis=1)

        c_edge = (mm(mean, mm(We1[416:480, :], We2v))
                  + mm(be1[:].reshape(1, H), We2v)
                  + be2[:].reshape(1, 1))
        c_node = (mm(mean, mm(Wn1[192:256, :], Wn2v))
                  + mm(bn1[:].reshape(1, H), Wn2v)
                  + bn2[:].reshape(1, NCLS))
        bias_s[:, :] = jnp.concatenate(
            [c_edge, jnp.zeros((1, 1), f32), c_node], axis=1)

    y = (mm(nf[:], w_s[0:128, :]) + mm(nh[:], w_s[128:192, :])
         + mm(cv[:], w_s[192:200, :]) + mm(te[:], w_s[200:208, :])
         + bias_s[:, :])

    sd_out[:, :] = y[:, 0:2]
    np_out[:, :] = y[:, 2:6]


def _node_stage(nf, nh, cv, te, We1, be1, We2, be2, Wn1, bn1, Wn2, bn2):
    full = lambda shape: pl.BlockSpec(shape, lambda i: (0, 0))
    return pl.pallas_call(
        _tc_body,
        grid=(G,),
        in_specs=[
            pl.BlockSpec((NB, D_FEAT), lambda i: (i, 0)),
            pl.BlockSpec((NB, D_HID), lambda i: (i, 0)),
            pl.BlockSpec((NB, D_COM), lambda i: (i, 0)),
            pl.BlockSpec((NB, D_TYPE), lambda i: (i, 0)),
            full((N, D_HID)),
            full((480, H)), pl.BlockSpec((H,), lambda i: (0,)),
            full((H, 1)), pl.BlockSpec((1,), lambda i: (0,)),
            full((272, H)), pl.BlockSpec((H,), lambda i: (0,)),
            full((H, NCLS)), pl.BlockSpec((NCLS,), lambda i: (0,)),
        ],
        out_specs=(
            pl.BlockSpec((NB, 2), lambda i: (i, 0)),
            pl.BlockSpec((NB, NCLS), lambda i: (i, 0)),
        ),
        out_shape=(
            jax.ShapeDtypeStruct((N, 2), jnp.float32),
            jax.ShapeDtypeStruct((N, NCLS), jnp.float32),
        ),
        scratch_shapes=[
            pltpu.VMEM((208, 6), jnp.float32),
            pltpu.VMEM((1, 6), jnp.float32),
        ],
    )(nf, nh, cv, te, nh, We1, be1, We2, be2, Wn1, bn1, Wn2, bn2)


def _sc_edge_body(sd_hbm, ei_hbm, out_hbm, sd_v, src_v, dst_v, out_v):
    wid = lax.axis_index("s") * NC + lax.axis_index("c")
    base = wid * E_PER_W
    pltpu.sync_copy(sd_hbm, sd_v)
    pltpu.sync_copy(ei_hbm.at[pl.ds(base, E_PER_W)], src_v)
    pltpu.sync_copy(ei_hbm.at[pl.ds(E + base, E_PER_W)], dst_v)

    col0 = jnp.zeros((LANES,), jnp.int32)
    col1 = jnp.ones((LANES,), jnp.int32)

    def body(i, carry):
        off = i * LANES
        si = src_v[pl.ds(off, LANES)]
        di = dst_v[pl.ds(off, LANES)]
        sv = plsc.load_gather(sd_v, [si, col0])
        dv = plsc.load_gather(sd_v, [di, col1])
        out_v[pl.ds(off, LANES)] = sv + dv
        return carry

    lax.fori_loop(0, E_PER_W // LANES, body, 0)
    pltpu.sync_copy(out_v, out_hbm.at[pl.ds(base, E_PER_W)])


@functools.lru_cache(maxsize=None)
def _edge_stage():
    return pl.kernel(
        _sc_edge_body,
        out_type=jax.ShapeDtypeStruct((E,), jnp.float32),
        mesh=plsc.VectorSubcoreMesh(core_axis_name="c",
                                    subcore_axis_name="s"),
        compiler_params=pltpu.CompilerParams(needs_layout_passes=False,
                                             use_tc_tiling_on_sc=False),
        scratch_types=[
            pltpu.VMEM((N, 2), jnp.float32),
            pltpu.VMEM((E_PER_W,), jnp.int32),
            pltpu.VMEM((E_PER_W,), jnp.int32),
            pltpu.VMEM((E_PER_W,), jnp.float32),
        ],
    )


def kernel(node_features, node_hidden, common_vars, type_emb, edge_index,
           We1, be1, We2, be2, Wn1, bn1, Wn2, bn2):
    sd, node_pred = _node_stage(
        node_features, node_hidden, common_vars, type_emb,
        We1, be1, We2, be2, Wn1, bn1, Wn2, bn2)
    edge_pred = _edge_stage()(sd, edge_index.reshape(2 * E))
    return (edge_pred, node_pred)


# speedup vs baseline: 1.1511x; 1.1511x over previous
import functools

import jax
import jax.numpy as jnp
from jax import lax
from jax.experimental import pallas as pl
from jax.experimental.pallas import tpu as pltpu
from jax.experimental.pallas import tpu_sc as plsc

N = 10000
E = 320000
D_FEAT = 128
D_HID = 64
D_COM = 8
D_TYPE = 8
H = 32
NCLS = 4

NC = 2
NS = 16
LANES = 16
NW = NC * NS
E_PER_W = E // NW


NB = 2000
G = N // NB


def _tc_body(nf, nh, cv, te, nh_full, We1, be1, We2, be2, Wn1, bn1, Wn2,
             bn2, sd_out, np_out, w_s, bias_s):
    f32 = jnp.float32

    def mm(a, b):
        return jax.lax.dot(a, b, precision=jax.lax.Precision.HIGHEST,
                           preferred_element_type=f32)

    @pl.when(pl.program_id(0) == 0)
    def _prep():
        We2v = We2[:]
        Wn2v = Wn2[:]
        mean = jnp.mean(nh_full[:], axis=0, keepdims=True)

        w_s[0:128, :] = jnp.concatenate(
            [mm(We1[0:128, :], We2v), mm(We1[208:336, :], We2v),
             mm(Wn1[0:128, :], Wn2v)], axis=1)
        w_s[128:192, :] = jnp.concatenate(
            [mm(We1[128:192, :], We2v), mm(We1[336:400, :], We2v),
             mm(Wn1[128:192, :], Wn2v)], axis=1)
        w_s[192:200, :] = jnp.concatenate(
            [mm(We1[192:200, :], We2v), mm(We1[200:208, :], We2v),
             mm(Wn1[264:272, :], Wn2v)], axis=1)
        w_s[200:208, :] = jnp.concatenate(
            [mm(We1[408:416, :], We2v), mm(We1[400:408, :], We2v),
             mm(Wn1[256:264, :], Wn2v)], axis=1)

        c_edge = (mm(mean, mm(We1[416:480, :], We2v))
                  + mm(be1[:].reshape(1, H), We2v)
                  + be2[:].reshape(1, 1))
        c_node = (mm(mean, mm(Wn1[192:256, :], Wn2v))
                  + mm(bn1[:].reshape(1, H), Wn2v)
                  + bn2[:].reshape(1, NCLS))
        bias_s[:, :] = jnp.concatenate(
            [c_edge, jnp.zeros((1, 1), f32), c_node], axis=1)

    y = (mm(nf[:], w_s[0:128, :]) + mm(nh[:], w_s[128:192, :])
         + mm(cv[:], w_s[192:200, :]) + mm(te[:], w_s[200:208, :])
         + bias_s[:, :])

    sd_out[:, :] = y[:, 0:2]
    np_out[:, :] = y[:, 2:6]


def _node_stage(nf, nh, cv, te, We1, be1, We2, be2, Wn1, bn1, Wn2, bn2):
    full = lambda shape: pl.BlockSpec(shape, lambda i: (0, 0))
    return pl.pallas_call(
        _tc_body,
        grid=(G,),
        in_specs=[
            pl.BlockSpec((NB, D_FEAT), lambda i: (i, 0)),
            pl.BlockSpec((NB, D_HID), lambda i: (i, 0)),
            pl.BlockSpec((NB, D_COM), lambda i: (i, 0)),
            pl.BlockSpec((NB, D_TYPE), lambda i: (i, 0)),
            full((N, D_HID)),
            full((480, H)), pl.BlockSpec((H,), lambda i: (0,)),
            full((H, 1)), pl.BlockSpec((1,), lambda i: (0,)),
            full((272, H)), pl.BlockSpec((H,), lambda i: (0,)),
            full((H, NCLS)), pl.BlockSpec((NCLS,), lambda i: (0,)),
        ],
        out_specs=(
            pl.BlockSpec((NB, 2), lambda i: (i, 0)),
            pl.BlockSpec((NB, NCLS), lambda i: (i, 0)),
        ),
        out_shape=(
            jax.ShapeDtypeStruct((N, 2), jnp.float32),
            jax.ShapeDtypeStruct((N, NCLS), jnp.float32),
        ),
        scratch_shapes=[
            pltpu.VMEM((208, 6), jnp.float32),
            pltpu.VMEM((1, 6), jnp.float32),
        ],
    )(nf, nh, cv, te, nh, We1, be1, We2, be2, Wn1, bn1, Wn2, bn2)


def _sc_edge_body(sd_hbm, ei_hbm, out_hbm, sd_v, src_v, dst_v, out_v):
    wid = lax.axis_index("s") * NC + lax.axis_index("c")
    base = wid * E_PER_W
    pltpu.sync_copy(sd_hbm, sd_v)
    pltpu.sync_copy(ei_hbm.at[pl.ds(base, E_PER_W)], src_v)
    pltpu.sync_copy(ei_hbm.at[pl.ds(E + base, E_PER_W)], dst_v)

    @plsc.parallel_loop(0, E_PER_W, LANES, unroll=5)
    def _body(off):
        si = src_v[pl.ds(off, LANES)] * 2
        di = dst_v[pl.ds(off, LANES)] * 2 + 1
        sv = plsc.load_gather(sd_v, [si])
        dv = plsc.load_gather(sd_v, [di])
        out_v[pl.ds(off, LANES)] = sv + dv

    pltpu.sync_copy(out_v, out_hbm.at[pl.ds(base, E_PER_W)])


@functools.lru_cache(maxsize=None)
def _edge_stage():
    return pl.kernel(
        _sc_edge_body,
        out_type=jax.ShapeDtypeStruct((E,), jnp.float32),
        mesh=plsc.VectorSubcoreMesh(core_axis_name="c",
                                    subcore_axis_name="s"),
        compiler_params=pltpu.CompilerParams(needs_layout_passes=False),
        scratch_types=[
            pltpu.VMEM((2 * N,), jnp.float32),
            pltpu.VMEM((E_PER_W,), jnp.int32),
            pltpu.VMEM((E_PER_W,), jnp.int32),
            pltpu.VMEM((E_PER_W,), jnp.float32),
        ],
    )


def kernel(node_features, node_hidden, common_vars, type_emb, edge_index,
           We1, be1, We2, be2, Wn1, bn1, Wn2, bn2):
    sd, node_pred = _node_stage(
        node_features, node_hidden, common_vars, type_emb,
        We1, be1, We2, be2, Wn1, bn1, Wn2, bn2)
    edge_pred = _edge_stage()(sd.reshape(2 * N), edge_index.reshape(2 * E))
    return (edge_pred, node_pred)


# speedup vs baseline: 1.1563x; 1.0045x over previous
import functools

import jax
import jax.numpy as jnp
from jax import lax
from jax.experimental import pallas as pl
from jax.experimental.pallas import tpu as pltpu
from jax.experimental.pallas import tpu_sc as plsc

N = 10000
E = 320000
D_FEAT = 128
D_HID = 64
D_COM = 8
D_TYPE = 8
H = 32
NCLS = 4

NC = 2
NS = 16
LANES = 16
NW = NC * NS
E_PER_W = E // NW


NB = 2048
G = (N + NB - 1) // NB
NPAD = G * NB


def _tc_body(nf, nh, cv, te, nh_full, We1, be1, We2, be2, Wn1, bn1, Wn2,
             bn2, sd_out, np_out, w_s, bias_s):
    f32 = jnp.float32

    def mm(a, b):
        return jax.lax.dot(a, b, precision=jax.lax.Precision.HIGHEST,
                           preferred_element_type=f32)

    @pl.when(pl.program_id(0) == 0)
    def _prep():
        We2v = We2[:]
        Wn2v = Wn2[:]
        mean = jnp.mean(nh_full[:], axis=0, keepdims=True)

        w_s[0:128, :] = jnp.concatenate(
            [mm(We1[0:128, :], We2v), mm(We1[208:336, :], We2v),
             mm(Wn1[0:128, :], Wn2v)], axis=1)
        w_s[128:192, :] = jnp.concatenate(
            [mm(We1[128:192, :], We2v), mm(We1[336:400, :], We2v),
             mm(Wn1[128:192, :], Wn2v)], axis=1)
        w_s[192:200, :] = jnp.concatenate(
            [mm(We1[192:200, :], We2v), mm(We1[200:208, :], We2v),
             mm(Wn1[264:272, :], Wn2v)], axis=1)
        w_s[200:208, :] = jnp.concatenate(
            [mm(We1[408:416, :], We2v), mm(We1[400:408, :], We2v),
             mm(Wn1[256:264, :], Wn2v)], axis=1)

        c_edge = (mm(mean, mm(We1[416:480, :], We2v))
                  + mm(be1[:].reshape(1, H), We2v)
                  + be2[:].reshape(1, 1))
        c_node = (mm(mean, mm(Wn1[192:256, :], Wn2v))
                  + mm(bn1[:].reshape(1, H), Wn2v)
                  + bn2[:].reshape(1, NCLS))
        bias_s[:, :] = jnp.concatenate(
            [c_edge, jnp.zeros((1, 1), f32), c_node], axis=1)

    y = (mm(nf[:], w_s[0:128, :]) + mm(nh[:], w_s[128:192, :])
         + mm(cv[:], w_s[192:200, :]) + mm(te[:], w_s[200:208, :])
         + bias_s[:, :])

    sd_out[:, :] = y[:, 0:2].T
    np_out[:, :] = y[:, 2:6]


def _node_stage(nf, nh, cv, te, We1, be1, We2, be2, Wn1, bn1, Wn2, bn2):
    full = lambda shape: pl.BlockSpec(shape, lambda i: (0, 0))
    return pl.pallas_call(
        _tc_body,
        grid=(G,),
        in_specs=[
            pl.BlockSpec((NB, D_FEAT), lambda i: (i, 0)),
            pl.BlockSpec((NB, D_HID), lambda i: (i, 0)),
            pl.BlockSpec((NB, D_COM), lambda i: (i, 0)),
            pl.BlockSpec((NB, D_TYPE), lambda i: (i, 0)),
            full((N, D_HID)),
            full((480, H)), pl.BlockSpec((H,), lambda i: (0,)),
            full((H, 1)), pl.BlockSpec((1,), lambda i: (0,)),
            full((272, H)), pl.BlockSpec((H,), lambda i: (0,)),
            full((H, NCLS)), pl.BlockSpec((NCLS,), lambda i: (0,)),
        ],
        out_specs=(
            pl.BlockSpec((2, NB), lambda i: (0, i)),
            pl.BlockSpec((NB, NCLS), lambda i: (i, 0)),
        ),
        out_shape=(
            jax.ShapeDtypeStruct((2, NPAD), jnp.float32),
            jax.ShapeDtypeStruct((N, NCLS), jnp.float32),
        ),
        scratch_shapes=[
            pltpu.VMEM((208, 6), jnp.float32),
            pltpu.VMEM((1, 6), jnp.float32),
        ],
    )(nf, nh, cv, te, nh, We1, be1, We2, be2, Wn1, bn1, Wn2, bn2)


def _sc_edge_body(sd_hbm, ei_hbm, out_hbm, sd_v, src_v, dst_v, out_v):
    wid = lax.axis_index("s") * NC + lax.axis_index("c")
    base = wid * E_PER_W
    pltpu.sync_copy(sd_hbm, sd_v)
    pltpu.sync_copy(ei_hbm.at[pl.ds(base, E_PER_W)], src_v)
    pltpu.sync_copy(ei_hbm.at[pl.ds(E + base, E_PER_W)], dst_v)

    @plsc.parallel_loop(0, E_PER_W, LANES, unroll=5)
    def _body(off):
        si = src_v[pl.ds(off, LANES)]
        di = dst_v[pl.ds(off, LANES)] + NPAD
        sv = plsc.load_gather(sd_v, [si])
        dv = plsc.load_gather(sd_v, [di])
        out_v[pl.ds(off, LANES)] = sv + dv

    pltpu.sync_copy(out_v, out_hbm.at[pl.ds(base, E_PER_W)])


@functools.lru_cache(maxsize=None)
def _edge_stage():
    return pl.kernel(
        _sc_edge_body,
        out_type=jax.ShapeDtypeStruct((E,), jnp.float32),
        mesh=plsc.VectorSubcoreMesh(core_axis_name="c",
                                    subcore_axis_name="s"),
        compiler_params=pltpu.CompilerParams(needs_layout_passes=False),
        scratch_types=[
            pltpu.VMEM((2 * NPAD,), jnp.float32),
            pltpu.VMEM((E_PER_W,), jnp.int32),
            pltpu.VMEM((E_PER_W,), jnp.int32),
            pltpu.VMEM((E_PER_W,), jnp.float32),
        ],
    )


def kernel(node_features, node_hidden, common_vars, type_emb, edge_index,
           We1, be1, We2, be2, Wn1, bn1, Wn2, bn2):
    sd, node_pred = _node_stage(
        node_features, node_hidden, common_vars, type_emb,
        We1, be1, We2, be2, Wn1, bn1, Wn2, bn2)
    edge_pred = _edge_stage()(sd.reshape(2 * NPAD),
                              edge_index.reshape(2 * E))
    return (edge_pred, node_pred)


# speedup vs baseline: 1.2178x; 1.0532x over previous
import functools

import jax
import jax.numpy as jnp
from jax import lax
from jax.experimental import pallas as pl
from jax.experimental.pallas import tpu as pltpu
from jax.experimental.pallas import tpu_sc as plsc

N = 10000
E = 320000
D_FEAT = 128
D_HID = 64
D_COM = 8
D_TYPE = 8
H = 32
NCLS = 4

NC = 2
NS = 16
LANES = 16
NW = NC * NS
E_PER_W = E // NW


NB = 2048
G = (N + NB - 1) // NB
NPAD = G * NB


def _tc_body(nf, x2, x2_full, We1, be1, We2, be2, Wn1, bn1, Wn2,
             bn2, sd_out, np_out, w_s, bias_s):
    f32 = jnp.float32

    def mm(a, b):
        return jax.lax.dot(a, b, precision=jax.lax.Precision.HIGHEST,
                           preferred_element_type=f32)

    @pl.when(pl.program_id(0) == 0)
    def _prep():
        We2v = We2[:]
        Wn2v = Wn2[:]
        colsum = jnp.sum(x2_full[:], axis=0, keepdims=True)
        mean = colsum[:, 0:D_HID] * (1.0 / N)

        w_s[0:128, :] = jnp.concatenate(
            [mm(We1[0:128, :], We2v), mm(We1[208:336, :], We2v),
             mm(Wn1[0:128, :], Wn2v)], axis=1)
        w_s[128:192, :] = jnp.concatenate(
            [mm(We1[128:192, :], We2v), mm(We1[336:400, :], We2v),
             mm(Wn1[128:192, :], Wn2v)], axis=1)
        w_s[192:200, :] = jnp.concatenate(
            [mm(We1[192:200, :], We2v), mm(We1[200:208, :], We2v),
             mm(Wn1[264:272, :], Wn2v)], axis=1)
        w_s[200:208, :] = jnp.concatenate(
            [mm(We1[408:416, :], We2v), mm(We1[400:408, :], We2v),
             mm(Wn1[256:264, :], Wn2v)], axis=1)

        c_edge = (mm(mean, mm(We1[416:480, :], We2v))
                  + mm(be1[:].reshape(1, H), We2v)
                  + be2[:].reshape(1, 1))
        c_node = (mm(mean, mm(Wn1[192:256, :], Wn2v))
                  + mm(bn1[:].reshape(1, H), Wn2v)
                  + bn2[:].reshape(1, NCLS))
        bias_s[:, :] = jnp.concatenate(
            [c_edge, jnp.zeros((1, 1), f32), c_node], axis=1)

    y = (mm(nf[:], w_s[0:128, :]) + mm(x2[:], w_s[128:208, :])
         + bias_s[:, :])

    sd_out[:, :] = y[:, 0:2].T
    np_out[:, :] = y[:, 2:6]


def _node_stage(nf, x2, We1, be1, We2, be2, Wn1, bn1, Wn2, bn2):
    full = lambda shape: pl.BlockSpec(shape, lambda i: (0, 0))
    return pl.pallas_call(
        _tc_body,
        grid=(G,),
        in_specs=[
            pl.BlockSpec((NB, D_FEAT), lambda i: (i, 0)),
            pl.BlockSpec((NB, 80), lambda i: (i, 0)),
            full((N, 80)),
            full((480, H)), pl.BlockSpec((H,), lambda i: (0,)),
            full((H, 1)), pl.BlockSpec((1,), lambda i: (0,)),
            full((272, H)), pl.BlockSpec((H,), lambda i: (0,)),
            full((H, NCLS)), pl.BlockSpec((NCLS,), lambda i: (0,)),
        ],
        out_specs=(
            pl.BlockSpec((2, NB), lambda i: (0, i)),
            pl.BlockSpec((NB, NCLS), lambda i: (i, 0)),
        ),
        out_shape=(
            jax.ShapeDtypeStruct((2, NPAD), jnp.float32),
            jax.ShapeDtypeStruct((N, NCLS), jnp.float32),
        ),
        scratch_shapes=[
            pltpu.VMEM((208, 6), jnp.float32),
            pltpu.VMEM((1, 6), jnp.float32),
        ],
    )(nf, x2, x2, We1, be1, We2, be2, Wn1, bn1, Wn2, bn2)


def _sc_edge_body(sd_hbm, ei_hbm, out_hbm, sd_v, src_v, dst_v, out_v):
    wid = lax.axis_index("s") * NC + lax.axis_index("c")
    base = wid * E_PER_W
    pltpu.sync_copy(sd_hbm, sd_v)
    pltpu.sync_copy(ei_hbm.at[pl.ds(base, E_PER_W)], src_v)
    pltpu.sync_copy(ei_hbm.at[pl.ds(E + base, E_PER_W)], dst_v)

    @plsc.parallel_loop(0, E_PER_W, LANES, unroll=25)
    def _body(off):
        si = src_v[pl.ds(off, LANES)]
        di = dst_v[pl.ds(off, LANES)] + NPAD
        sv = plsc.load_gather(sd_v, [si])
        dv = plsc.load_gather(sd_v, [di])
        out_v[pl.ds(off, LANES)] = sv + dv

    pltpu.sync_copy(out_v, out_hbm.at[pl.ds(base, E_PER_W)])


@functools.lru_cache(maxsize=None)
def _edge_stage():
    return pl.kernel(
        _sc_edge_body,
        out_type=jax.ShapeDtypeStruct((E,), jnp.float32),
        mesh=plsc.VectorSubcoreMesh(core_axis_name="c",
                                    subcore_axis_name="s"),
        compiler_params=pltpu.CompilerParams(needs_layout_passes=False),
        scratch_types=[
            pltpu.VMEM((2 * NPAD,), jnp.float32),
            pltpu.VMEM((E_PER_W,), jnp.int32),
            pltpu.VMEM((E_PER_W,), jnp.int32),
            pltpu.VMEM((E_PER_W,), jnp.float32),
        ],
    )


def kernel(node_features, node_hidden, common_vars, type_emb, edge_index,
           We1, be1, We2, be2, Wn1, bn1, Wn2, bn2):
    x2 = jnp.concatenate([node_hidden, common_vars, type_emb], axis=1)
    sd, node_pred = _node_stage(
        node_features, x2, We1, be1, We2, be2, Wn1, bn1, Wn2, bn2)
    edge_pred = _edge_stage()(sd.reshape(2 * NPAD),
                              edge_index.reshape(2 * E))
    return (edge_pred, node_pred)


# speedup vs baseline: 1.2396x; 1.0179x over previous
import functools

import jax
import jax.numpy as jnp
from jax import lax
from jax.experimental import pallas as pl
from jax.experimental.pallas import tpu as pltpu
from jax.experimental.pallas import tpu_sc as plsc

N = 10000
E = 320000
D_FEAT = 128
D_HID = 64
D_COM = 8
D_TYPE = 8
H = 32
NCLS = 4

NC = 2
NS = 16
LANES = 16
NW = NC * NS
E_PER_W = E // NW


NB = 2048
G = (N + NB - 1) // NB
NPAD = G * NB


def _tc_body(nf, x2, We1, be1, We2, be2, Wn1, bn1, Wn2,
             bn2, sd_out, np_out, w_s, colsum_s, sd_s):
    f32 = jnp.float32
    i = pl.program_id(0)

    def mm(a, b):
        return jax.lax.dot(a, b, precision=jax.lax.Precision.HIGHEST,
                           preferred_element_type=f32)

    @pl.when(i == 0)
    def _prep():
        We2v = We2[:]
        Wn2v = Wn2[:]
        w_s[0:128, :] = jnp.concatenate(
            [mm(We1[0:128, :], We2v), mm(We1[208:336, :], We2v),
             mm(Wn1[0:128, :], Wn2v)], axis=1)
        w_s[128:192, :] = jnp.concatenate(
            [mm(We1[128:192, :], We2v), mm(We1[336:400, :], We2v),
             mm(Wn1[128:192, :], Wn2v)], axis=1)
        w_s[192:200, :] = jnp.concatenate(
            [mm(We1[192:200, :], We2v), mm(We1[200:208, :], We2v),
             mm(Wn1[264:272, :], Wn2v)], axis=1)
        w_s[200:208, :] = jnp.concatenate(
            [mm(We1[408:416, :], We2v), mm(We1[400:408, :], We2v),
             mm(Wn1[256:264, :], Wn2v)], axis=1)
        colsum_s[:, :] = jnp.zeros((1, 80), f32)

    x2v = x2[:]
    y = mm(nf[:], w_s[0:128, :]) + mm(x2v, w_s[128:208, :])
    sd_s[:, pl.ds(i * NB, NB)] = y[:, 0:2].T

    valid = (lax.broadcasted_iota(jnp.int32, (NB, 1), 0) + i * NB) < N
    colsum_s[:, :] = colsum_s[:, :] + jnp.sum(
        jnp.where(valid, x2v, 0.0), axis=0, keepdims=True)

    @pl.when(i < G - 1)
    def _wr():
        np_out[pl.ds(i * NB, NB), :] = y[:, 2:6]

    @pl.when(i == G - 1)
    def _finish():
        np_out[pl.ds((G - 1) * NB, N - (G - 1) * NB), :] = (
            y[0:N - (G - 1) * NB, 2:6])
        We2v = We2[:]
        Wn2v = Wn2[:]
        mean = colsum_s[:, 0:D_HID] * (1.0 / N)
        c_edge = (mm(mean, mm(We1[416:480, :], We2v))
                  + mm(be1[:].reshape(1, H), We2v)
                  + be2[:].reshape(1, 1))
        c_node = (mm(mean, mm(Wn1[192:256, :], Wn2v))
                  + mm(bn1[:].reshape(1, H), Wn2v)
                  + bn2[:].reshape(1, NCLS))
        sd_bias = jnp.concatenate(
            [c_edge, jnp.zeros((1, 1), jnp.float32)], axis=0)
        sd_out[:, :] = sd_s[:, :] + sd_bias
        np_out[:, :] = np_out[:, :] + c_node


def _node_stage(nf, x2, We1, be1, We2, be2, Wn1, bn1, Wn2, bn2):
    full = lambda shape: pl.BlockSpec(shape, lambda i: (0, 0))
    return pl.pallas_call(
        _tc_body,
        grid=(G,),
        in_specs=[
            pl.BlockSpec((NB, D_FEAT), lambda i: (i, 0)),
            pl.BlockSpec((NB, 80), lambda i: (i, 0)),
            full((480, H)), pl.BlockSpec((H,), lambda i: (0,)),
            full((H, 1)), pl.BlockSpec((1,), lambda i: (0,)),
            full((272, H)), pl.BlockSpec((H,), lambda i: (0,)),
            full((H, NCLS)), pl.BlockSpec((NCLS,), lambda i: (0,)),
        ],
        out_specs=(
            full((2, NPAD)),
            full((N, NCLS)),
        ),
        out_shape=(
            jax.ShapeDtypeStruct((2, NPAD), jnp.float32),
            jax.ShapeDtypeStruct((N, NCLS), jnp.float32),
        ),
        scratch_shapes=[
            pltpu.VMEM((208, 6), jnp.float32),
            pltpu.VMEM((1, 80), jnp.float32),
            pltpu.VMEM((2, NPAD), jnp.float32),
        ],
    )(nf, x2, We1, be1, We2, be2, Wn1, bn1, Wn2, bn2)


def _sc_edge_body(sd_hbm, ei_hbm, out_hbm, sd_v, src_v, dst_v, out_v):
    wid = lax.axis_index("s") * NC + lax.axis_index("c")
    base = wid * E_PER_W
    pltpu.sync_copy(sd_hbm, sd_v)
    pltpu.sync_copy(ei_hbm.at[pl.ds(base, E_PER_W)], src_v)
    pltpu.sync_copy(ei_hbm.at[pl.ds(E + base, E_PER_W)], dst_v)

    @plsc.parallel_loop(0, E_PER_W, LANES, unroll=25)
    def _body(off):
        si = src_v[pl.ds(off, LANES)]
        di = dst_v[pl.ds(off, LANES)] + NPAD
        sv = plsc.load_gather(sd_v, [si])
        dv = plsc.load_gather(sd_v, [di])
        out_v[pl.ds(off, LANES)] = sv + dv

    pltpu.sync_copy(out_v, out_hbm.at[pl.ds(base, E_PER_W)])


@functools.lru_cache(maxsize=None)
def _edge_stage():
    return pl.kernel(
        _sc_edge_body,
        out_type=jax.ShapeDtypeStruct((E,), jnp.float32),
        mesh=plsc.VectorSubcoreMesh(core_axis_name="c",
                                    subcore_axis_name="s"),
        compiler_params=pltpu.CompilerParams(needs_layout_passes=False),
        scratch_types=[
            pltpu.VMEM((2 * NPAD,), jnp.float32),
            pltpu.VMEM((E_PER_W,), jnp.int32),
            pltpu.VMEM((E_PER_W,), jnp.int32),
            pltpu.VMEM((E_PER_W,), jnp.float32),
        ],
    )


def kernel(node_features, node_hidden, common_vars, type_emb, edge_index,
           We1, be1, We2, be2, Wn1, bn1, Wn2, bn2):
    x2 = jnp.concatenate([node_hidden, common_vars, type_emb], axis=1)
    sd, node_pred = _node_stage(
        node_features, x2, We1, be1, We2, be2, Wn1, bn1, Wn2, bn2)
    edge_pred = _edge_stage()(sd.reshape(2 * NPAD),
                              edge_index.reshape(2 * E))
    return (edge_pred, node_pred)


# speedup vs baseline: 1.3411x; 1.0818x over previous
import functools

import jax
import jax.numpy as jnp
from jax import lax
from jax.experimental import pallas as pl
from jax.experimental.pallas import tpu as pltpu
from jax.experimental.pallas import tpu_sc as plsc

N = 10000
E = 320000
D_FEAT = 128
D_HID = 64
D_COM = 8
D_TYPE = 8
H = 32
NCLS = 4

NC = 2
NS = 16
LANES = 16
NW = NC * NS
E_PER_W = E // NW


NB = 2048
G = (N + NB - 1) // NB
NPAD = G * NB


def _tc_body(nf, x2, We1, be1, We2, be2, Wn1, bn1, Wn2,
             bn2, sd_out, np_out, w_s, colsum_s, sd_s):
    f32 = jnp.float32
    i = pl.program_id(0)

    def mm(a, b):
        return jax.lax.dot(a, b, precision=jax.lax.Precision.HIGHEST,
                           preferred_element_type=f32)

    @pl.when(i == 0)
    def _prep():
        We2v = We2[:]
        Wn2v = Wn2[:]
        w_s[0:128, :] = jnp.concatenate(
            [mm(We1[0:128, :], We2v), mm(We1[208:336, :], We2v),
             mm(Wn1[0:128, :], Wn2v)], axis=1)
        w_s[128:192, :] = jnp.concatenate(
            [mm(We1[128:192, :], We2v), mm(We1[336:400, :], We2v),
             mm(Wn1[128:192, :], Wn2v)], axis=1)
        w_s[192:200, :] = jnp.concatenate(
            [mm(We1[192:200, :], We2v), mm(We1[200:208, :], We2v),
             mm(Wn1[264:272, :], Wn2v)], axis=1)
        w_s[200:208, :] = jnp.concatenate(
            [mm(We1[408:416, :], We2v), mm(We1[400:408, :], We2v),
             mm(Wn1[256:264, :], Wn2v)], axis=1)
        colsum_s[:, :] = jnp.zeros((1, 80), f32)

    x2v = x2[:]
    y = mm(nf[:], w_s[0:128, :]) + mm(x2v, w_s[128:208, :])
    sd_s[:, pl.ds(i * NB, NB)] = y[:, 0:2].T

    valid = (lax.broadcasted_iota(jnp.int32, (NB, 1), 0) + i * NB) < N
    colsum_s[:, :] = colsum_s[:, :] + jnp.sum(
        jnp.where(valid, x2v, 0.0), axis=0, keepdims=True)

    @pl.when(i < G - 1)
    def _wr():
        np_out[pl.ds(i * NB, NB), :] = y[:, 2:6]

    @pl.when(i == G - 1)
    def _finish():
        np_out[pl.ds((G - 1) * NB, N - (G - 1) * NB), :] = (
            y[0:N - (G - 1) * NB, 2:6])
        We2v = We2[:]
        Wn2v = Wn2[:]
        mean = colsum_s[:, 0:D_HID] * (1.0 / N)
        c_edge = (mm(mean, mm(We1[416:480, :], We2v))
                  + mm(be1[:].reshape(1, H), We2v)
                  + be2[:].reshape(1, 1))
        c_node = (mm(mean, mm(Wn1[192:256, :], Wn2v))
                  + mm(bn1[:].reshape(1, H), Wn2v)
                  + bn2[:].reshape(1, NCLS))
        sd_bias = jnp.concatenate(
            [c_edge, jnp.zeros((1, 1), jnp.float32)], axis=0)
        sd_out[:, :] = sd_s[:, :] + sd_bias
        np_out[:, :] = np_out[:, :] + c_node


def _node_stage(nf, x2, We1, be1, We2, be2, Wn1, bn1, Wn2, bn2):
    full = lambda shape: pl.BlockSpec(shape, lambda i: (0, 0))
    return pl.pallas_call(
        _tc_body,
        grid=(G,),
        in_specs=[
            pl.BlockSpec((NB, D_FEAT), lambda i: (i, 0)),
            pl.BlockSpec((NB, 80), lambda i: (i, 0)),
            full((480, H)), pl.BlockSpec((H,), lambda i: (0,)),
            full((H, 1)), pl.BlockSpec((1,), lambda i: (0,)),
            full((272, H)), pl.BlockSpec((H,), lambda i: (0,)),
            full((H, NCLS)), pl.BlockSpec((NCLS,), lambda i: (0,)),
        ],
        out_specs=(
            full((2, NPAD)),
            full((N, NCLS)),
        ),
        out_shape=(
            jax.ShapeDtypeStruct((2, NPAD), jnp.float32),
            jax.ShapeDtypeStruct((N, NCLS), jnp.float32),
        ),
        scratch_shapes=[
            pltpu.VMEM((208, 6), jnp.float32),
            pltpu.VMEM((1, 80), jnp.float32),
            pltpu.VMEM((2, NPAD), jnp.float32),
        ],
    )(nf, x2, We1, be1, We2, be2, Wn1, bn1, Wn2, bn2)


CW = 10240
CWL = E - 31 * CW


def _sc_edge_body(sd_hbm, ei_hbm, out_hbm, sd_v, ei_v, out_v):
    wid = lax.axis_index("s") * NC + lax.axis_index("c")
    base = wid * CW
    pltpu.sync_copy(sd_hbm, sd_v)

    def run(chunk, unroll):
        pltpu.sync_copy(ei_hbm.at[:, pl.ds(base, chunk)],
                        ei_v.at[:, pl.ds(0, chunk)])

        @plsc.parallel_loop(0, chunk, LANES, unroll=unroll)
        def _body(off):
            si = ei_v[0, pl.ds(off, LANES)]
            di = ei_v[1, pl.ds(off, LANES)] + NPAD
            sv = plsc.load_gather(sd_v, [si])
            dv = plsc.load_gather(sd_v, [di])
            out_v[pl.ds(off, LANES)] = sv + dv

        pltpu.sync_copy(out_v.at[pl.ds(0, chunk)],
                        out_hbm.at[pl.ds(base, chunk)])

    @pl.when(wid < NW - 1)
    def _main():
        run(CW, 20)

    @pl.when(wid == NW - 1)
    def _tail():
        run(CWL, 20)


@functools.lru_cache(maxsize=None)
def _edge_stage():
    return pl.kernel(
        _sc_edge_body,
        out_type=jax.ShapeDtypeStruct((E,), jnp.float32),
        mesh=plsc.VectorSubcoreMesh(core_axis_name="c",
                                    subcore_axis_name="s"),
        compiler_params=pltpu.CompilerParams(needs_layout_passes=False),
        scratch_types=[
            pltpu.VMEM((2 * NPAD,), jnp.float32),
            pltpu.VMEM((2, CW), jnp.int32),
            pltpu.VMEM((CW,), jnp.float32),
        ],
    )


def kernel(node_features, node_hidden, common_vars, type_emb, edge_index,
           We1, be1, We2, be2, Wn1, bn1, Wn2, bn2):
    x2 = jnp.concatenate([node_hidden, common_vars, type_emb], axis=1)
    sd, node_pred = _node_stage(
        node_features, x2, We1, be1, We2, be2, Wn1, bn1, Wn2, bn2)
    edge_pred = _edge_stage()(sd.reshape(2 * NPAD), edge_index)
    return (edge_pred, node_pred)
